# baseline (speedup 1.0000x reference)
import functools

import jax
import jax.numpy as jnp
from jax import lax
from jax.experimental import pallas as pl
from jax.experimental.pallas import tpu as pltpu
from jax.experimental.pallas import tpu_sc as plsc

VOCAB = 100000
EMBED = 64
BATCH = 4096

NUM_CORES = 2
NUM_SUBCORES = 16
NUM_WORKERS = NUM_CORES * NUM_SUBCORES
B_PER_W = BATCH // NUM_WORKERS
LANES = 16


def _sc_body(iw_hbm, tw_hbm, we_hbm, wt_hbm, b_hbm, bt_hbm,
             e_out, t_out, bi_out, bt_out,
             idx_i, idx_t, sem):
    wid = lax.axis_index("s") * NUM_CORES + lax.axis_index("c")
    base = wid * B_PER_W

    pltpu.sync_copy(iw_hbm.at[pl.ds(base, B_PER_W)], idx_i)
    pltpu.sync_copy(tw_hbm.at[pl.ds(base, B_PER_W)], idx_t)

    def fire(g, carry):
        vi = idx_i[pl.ds(g * LANES, LANES)]
        vt = idx_t[pl.ds(g * LANES, LANES)]
        for j in range(LANES):
            k = base + g * LANES + j
            ri = vi[j]
            rt = vt[j]
            pltpu.make_async_copy(
                we_hbm.at[pl.ds(ri, 1)], e_out.at[pl.ds(k, 1)], sem).start()
            pltpu.make_async_copy(
                wt_hbm.at[pl.ds(rt, 1)], t_out.at[pl.ds(k, 1)], sem).start()
            pltpu.make_async_copy(
                b_hbm.at[pl.ds(ri, 1)], bi_out.at[pl.ds(k, 1)], sem).start()
            pltpu.make_async_copy(
                bt_hbm.at[pl.ds(rt, 1)], bt_out.at[pl.ds(k, 1)], sem).start()
        return carry

    lax.fori_loop(0, B_PER_W // LANES, fire, 0)

    def drain(g, carry):
        pltpu.make_async_copy(
            we_hbm.at[pl.ds(0, 1)], e_out.at[pl.ds(0, 1)], sem).wait()
        pltpu.make_async_copy(
            wt_hbm.at[pl.ds(0, 1)], t_out.at[pl.ds(0, 1)], sem).wait()
        pltpu.make_async_copy(
            b_hbm.at[pl.ds(0, 1)], bi_out.at[pl.ds(0, 1)], sem).wait()
        pltpu.make_async_copy(
            bt_hbm.at[pl.ds(0, 1)], bt_out.at[pl.ds(0, 1)], sem).wait()
        return carry

    lax.fori_loop(0, B_PER_W, drain, 0)


_sc_gather_rows = functools.partial(
    pl.kernel,
    out_type=(
        jax.ShapeDtypeStruct((BATCH, EMBED), jnp.float32),
        jax.ShapeDtypeStruct((BATCH, EMBED), jnp.float32),
        jax.ShapeDtypeStruct((BATCH, 1), jnp.float32),
        jax.ShapeDtypeStruct((BATCH, 1), jnp.float32),
    ),
    mesh=plsc.VectorSubcoreMesh(core_axis_name="c", subcore_axis_name="s"),
    compiler_params=pltpu.CompilerParams(use_tc_tiling_on_sc=True),
    scratch_types=[
        pltpu.VMEM((B_PER_W,), jnp.int32),
        pltpu.VMEM((B_PER_W,), jnp.int32),
        pltpu.SemaphoreType.DMA,
    ],
)(_sc_body)


def _dot_body(e_ref, t_ref, bi_ref, bt_ref, dot_ref, bsum_ref):
    dot_ref[...] = jnp.sum(e_ref[...] * t_ref[...], axis=1, keepdims=True)
    bsum_ref[...] = bi_ref[...] + bt_ref[...]


@jax.jit
def _dot_bsum(e, t, bi, bt):
    return pl.pallas_call(
        _dot_body,
        out_shape=(
            jax.ShapeDtypeStruct((BATCH, 1), jnp.float32),
            jax.ShapeDtypeStruct((BATCH, 1), jnp.float32),
        ),
    )(e, t, bi, bt)


def _tc_body(bsum_ref, dot_ref, out_ref):
    out_ref[...] = bsum_ref[...] + dot_ref[...]


_BM = 256


@jax.jit
def _broadcast_add(bsum, dot):
    return pl.pallas_call(
        _tc_body,
        grid=(BATCH // _BM,),
        in_specs=[
            pl.BlockSpec((_BM, 1), lambda i: (i, 0)),
            pl.BlockSpec((1, BATCH), lambda i: (0, 0)),
        ],
        out_specs=pl.BlockSpec((_BM, BATCH), lambda i: (i, 0)),
        out_shape=jax.ShapeDtypeStruct((BATCH, BATCH), jnp.float32),
        compiler_params=pltpu.CompilerParams(
            dimension_semantics=("arbitrary",),
        ),
    )(bsum, dot)


@jax.jit
def kernel(input_word, target_word, W_embed, W_tilda, b, b_tilda):
    iw = input_word.astype(jnp.int32)
    tw = target_word.astype(jnp.int32)
    e, t, bi, bt = _sc_gather_rows(iw, tw, W_embed, W_tilda, b, b_tilda)
    dot, bsum = _dot_bsum(e, t, bi, bt)
    return _broadcast_add(bsum, dot.reshape(1, BATCH))


# speedup vs baseline: 2.5049x; 2.5049x over previous
import functools

import jax
import jax.numpy as jnp
from jax import lax
from jax.experimental import pallas as pl
from jax.experimental.pallas import tpu as pltpu
from jax.experimental.pallas import tpu_sc as plsc

VOCAB = 100000
EMBED = 64
BATCH = 4096

NUM_CORES = 2
NUM_SUBCORES = 16
NUM_WORKERS = NUM_CORES * NUM_SUBCORES
B_PER_W = BATCH // NUM_WORKERS
LANES = 16


def _sc_body(iw_hbm, tw_hbm, we_hbm, wt_hbm, b_hbm, bt_hbm,
             dot_hbm, bsum_hbm,
             idx_i, idx_t, e_v, t_v, bi_v, bt_v, dot_v, bsum_v, sem):
    wid = lax.axis_index("s") * NUM_CORES + lax.axis_index("c")
    base = wid * B_PER_W

    pltpu.sync_copy(iw_hbm.at[pl.ds(base, B_PER_W)], idx_i)
    pltpu.sync_copy(tw_hbm.at[pl.ds(base, B_PER_W)], idx_t)

    c0 = pltpu.async_copy(we_hbm.at[idx_i], e_v, sem)
    c1 = pltpu.async_copy(wt_hbm.at[idx_t], t_v, sem)
    c2 = pltpu.async_copy(b_hbm.at[idx_i], bi_v, sem)
    c3 = pltpu.async_copy(bt_hbm.at[idx_t], bt_v, sem)
    c0.wait()
    c1.wait()
    c2.wait()
    c3.wait()

    lane = lax.iota(jnp.int32, LANES)
    for g in range(B_PER_W // LANES):
        s = pl.ds(g * LANES, LANES)
        row_idx = g * LANES + lane

        def col(c, acc, row_idx=row_idx):
            cb = jnp.full((LANES,), c, jnp.int32)
            ev = plsc.load_gather(e_v, [row_idx, cb])
            tv = plsc.load_gather(t_v, [row_idx, cb])
            return acc + ev * tv

        dot_v[s] = lax.fori_loop(0, EMBED, col, jnp.zeros((LANES,), jnp.float32))
        bsum_v[s] = bi_v[s] + bt_v[s]

    pltpu.sync_copy(dot_v, dot_hbm.at[pl.ds(base, B_PER_W)])
    pltpu.sync_copy(bsum_v, bsum_hbm.at[pl.ds(base, B_PER_W)])


_sc_gather_dot = functools.partial(
    pl.kernel,
    out_type=(
        jax.ShapeDtypeStruct((BATCH,), jnp.float32),
        jax.ShapeDtypeStruct((BATCH,), jnp.float32),
    ),
    mesh=plsc.VectorSubcoreMesh(core_axis_name="c", subcore_axis_name="s"),
    compiler_params=pltpu.CompilerParams(
        needs_layout_passes=False, use_tc_tiling_on_sc=False),
    scratch_types=[
        pltpu.VMEM((B_PER_W,), jnp.int32),
        pltpu.VMEM((B_PER_W,), jnp.int32),
        pltpu.VMEM((B_PER_W, EMBED), jnp.float32),
        pltpu.VMEM((B_PER_W, EMBED), jnp.float32),
        pltpu.VMEM((B_PER_W,), jnp.float32),
        pltpu.VMEM((B_PER_W,), jnp.float32),
        pltpu.VMEM((B_PER_W,), jnp.float32),
        pltpu.VMEM((B_PER_W,), jnp.float32),
        pltpu.SemaphoreType.DMA,
    ],
)(_sc_body)


def _tc_body(bsum_ref, dot_ref, out_ref):
    out_ref[...] = bsum_ref[...] + dot_ref[...]


_BM = 256


@jax.jit
def _broadcast_add(bsum, dot):
    return pl.pallas_call(
        _tc_body,
        grid=(BATCH // _BM,),
        in_specs=[
            pl.BlockSpec((_BM, 1), lambda i: (i, 0)),
            pl.BlockSpec((1, BATCH), lambda i: (0, 0)),
        ],
        out_specs=pl.BlockSpec((_BM, BATCH), lambda i: (i, 0)),
        out_shape=jax.ShapeDtypeStruct((BATCH, BATCH), jnp.float32),
        compiler_params=pltpu.CompilerParams(
            dimension_semantics=("arbitrary",),
        ),
    )(bsum, dot)


@jax.jit
def kernel(input_word, target_word, W_embed, W_tilda, b, b_tilda):
    iw = input_word.astype(jnp.int32)
    tw = target_word.astype(jnp.int32)
    dot, bsum = _sc_gather_dot(iw, tw, W_embed, W_tilda,
                               jnp.sum(b, axis=1), jnp.sum(b_tilda, axis=1))
    return _broadcast_add(bsum.reshape(BATCH, 1), dot.reshape(1, BATCH))


# speedup vs baseline: 2.6437x; 1.0554x over previous
import functools

import jax
import jax.numpy as jnp
from jax import lax
from jax.experimental import pallas as pl
from jax.experimental.pallas import tpu as pltpu
from jax.experimental.pallas import tpu_sc as plsc

VOCAB = 100000
EMBED = 64
BATCH = 4096

NUM_CORES = 2
NUM_SUBCORES = 16
NUM_WORKERS = NUM_CORES * NUM_SUBCORES
B_PER_W = BATCH // NUM_WORKERS
LANES = 16
PACKED = 2 * EMBED

def _sc_body(iw_hbm, tw_hbm, we_hbm, wt_hbm, b_hbm, bt_hbm,
             dot_hbm, bsum_hbm,
             idx_i, idx_t, e_v, t_v, bi_v, bt_v,
             dot_v, bsum_v, sem):
    wid = lax.axis_index("s") * NUM_CORES + lax.axis_index("c")
    base = wid * B_PER_W

    pltpu.sync_copy(iw_hbm.at[pl.ds(base, B_PER_W)], idx_i)
    pltpu.sync_copy(tw_hbm.at[pl.ds(base, B_PER_W)], idx_t)

    c0 = pltpu.async_copy(we_hbm.at[idx_i], e_v, sem)
    c1 = pltpu.async_copy(wt_hbm.at[idx_t], t_v, sem)
    c2 = pltpu.async_copy(b_hbm.at[idx_i], bi_v, sem)
    c3 = pltpu.async_copy(bt_hbm.at[idx_t], bt_v, sem)
    c0.wait()
    c1.wait()
    c2.wait()
    c3.wait()

    lane = lax.iota(jnp.int32, LANES)
    for g in range(B_PER_W // LANES):
        s = pl.ds(g * LANES, LANES)
        row_idx = g * LANES + lane

        def col(c, acc, row_idx=row_idx):
            cb = jnp.full((LANES,), c, jnp.int32)
            ev = plsc.load_gather(e_v, [row_idx, cb])
            tv = plsc.load_gather(t_v, [row_idx, cb])
            return acc + ev * tv

        dot_v[s] = lax.fori_loop(0, EMBED, col, jnp.zeros((LANES,), jnp.float32))
        bsum_v[s] = bi_v[s] + bt_v[s]

    pltpu.sync_copy(dot_v, dot_hbm.at[pl.ds(base, B_PER_W)])
    pltpu.sync_copy(bsum_v, bsum_hbm.at[pl.ds(base, B_PER_W)])


_sc_gather_dot = functools.partial(
    pl.kernel,
    out_type=(
        jax.ShapeDtypeStruct((BATCH,), jnp.float32),
        jax.ShapeDtypeStruct((BATCH,), jnp.float32),
    ),
    mesh=plsc.VectorSubcoreMesh(core_axis_name="c", subcore_axis_name="s"),
    compiler_params=pltpu.CompilerParams(
        needs_layout_passes=False, use_tc_tiling_on_sc=True),
    scratch_types=[
        pltpu.VMEM((B_PER_W,), jnp.int32),
        pltpu.VMEM((B_PER_W,), jnp.int32),
        pltpu.VMEM((B_PER_W, PACKED), jnp.float32),
        pltpu.VMEM((B_PER_W, PACKED), jnp.float32),
        pltpu.VMEM((B_PER_W,), jnp.float32),
        pltpu.VMEM((B_PER_W,), jnp.float32),
        pltpu.VMEM((B_PER_W,), jnp.float32),
        pltpu.VMEM((B_PER_W,), jnp.float32),
        pltpu.SemaphoreType.DMA,
    ],
)(_sc_body)


def _tc_body(bsum_ref, dot_ref, out_ref):
    out_ref[...] = bsum_ref[...] + dot_ref[...]


_BM = 256


@jax.jit
def _broadcast_add(bsum, dot):
    return pl.pallas_call(
        _tc_body,
        grid=(BATCH // _BM,),
        in_specs=[
            pl.BlockSpec((_BM, 1), lambda i: (i, 0)),
            pl.BlockSpec((1, BATCH), lambda i: (0, 0)),
        ],
        out_specs=pl.BlockSpec((_BM, BATCH), lambda i: (i, 0)),
        out_shape=jax.ShapeDtypeStruct((BATCH, BATCH), jnp.float32),
        compiler_params=pltpu.CompilerParams(
            dimension_semantics=("arbitrary",),
        ),
    )(bsum, dot)


@jax.jit
def kernel(input_word, target_word, W_embed, W_tilda, b, b_tilda):
    iw = input_word.astype(jnp.int32)
    tw = target_word.astype(jnp.int32)
    we2 = jnp.pad(W_embed, ((0, 0), (0, PACKED - EMBED)))
    wt2 = jnp.pad(W_tilda, ((0, 0), (0, PACKED - EMBED)))
    dot, bsum = _sc_gather_dot(iw, tw, we2, wt2,
                               jnp.sum(b, axis=1), jnp.sum(b_tilda, axis=1))
    return _broadcast_add(bsum.reshape(BATCH, 1), dot.reshape(1, BATCH))
